# baseline (speedup 1.0000x reference)
import jax
import jax.numpy as jnp
from jax import lax
from jax.experimental import pallas as pl
from jax.experimental.pallas import tpu as pltpu
from jax.experimental.pallas import tpu_sc as plsc

NUM_SEGMENTS = 512
FEATS = 128
LANES = 16
NCHUNK = FEATS // LANES
NUM_CORES = 2
NUM_SUBCORES = 16
NUM_WORKERS = NUM_CORES * NUM_SUBCORES
SEGS_PER_WORKER = NUM_SEGMENTS // NUM_WORKERS
TILE = 224
NBUF = 4
NBOUNDS = 544


def _sc_body(feats_hbm, w_hbm, b_hbm, bounds_hbm, out_sum_hbm, out_max_hbm,
             fbuf0, fbuf1, fbuf2, fbuf3, wbuf, bbuf, bndbuf, tsum, tmax,
             sf0, sf1, sf2, sf3):
    n_rows = feats_hbm.shape[0]
    wid = lax.axis_index("c") * NUM_SUBCORES + lax.axis_index("s")
    seg_base = wid * SEGS_PER_WORKER

    pltpu.sync_copy(w_hbm, wbuf)
    pltpu.sync_copy(b_hbm, bbuf)
    pltpu.sync_copy(bounds_hbm, bndbuf)

    wv = [wbuf[pl.ds(k * LANES, LANES)] for k in range(NCHUNK)]
    bv = bbuf[...]

    def seg_bound(j):
        return bndbuf[pl.ds(j, LANES)][0]

    zero = jnp.zeros((LANES,), jnp.float32)
    ninf = jnp.full((LANES,), -jnp.inf, jnp.float32)

    def init_row(i, _):
        for k in range(NCHUNK):
            tsum[i, pl.ds(k * LANES, LANES)] = zero
            tmax[i, pl.ds(k * LANES, LANES)] = ninf
        return 0

    lax.fori_loop(0, SEGS_PER_WORKER, init_row, 0)

    r0 = seg_bound(seg_base)
    r1 = seg_bound(seg_base + SEGS_PER_WORKER)
    r0a = (r0 // 8) * 8
    ntiles = lax.div(r1 - r0a + TILE - 1, TILE)

    def tile_start(t):
        start = r0a + t * TILE
        return start, jnp.minimum(start, n_rows - TILE)

    def feat_copy(t, fb, sf):
        _, start_c = tile_start(t)
        return pltpu.make_async_copy(
            feats_hbm.at[pl.ds(start_c, TILE)], fb, sf)

    def gate_of(d_u):
        z = jnp.sum(d_u) + bv
        return 1.0 / (1.0 + jnp.exp(-z))

    def process_tile(t, fbuf, carry):
        start, start_c = tile_start(t)
        a = jnp.maximum(r0, start)
        b = jnp.minimum(r1, start + TILE)

        def row_at(i, c):
            acc, mx = c
            x = [fbuf[i, pl.ds(k * LANES, LANES)] for k in range(NCHUNK)]
            d = x[0] * wv[0]
            for k in range(1, NCHUNK):
                d = d + x[k] * wv[k]
            g = gate_of(d)
            nacc = tuple(acc[k] + x[k] * g for k in range(NCHUNK))
            nmx = tuple(jnp.maximum(mx[k], x[k]) for k in range(NCHUNK))
            return nacc, nmx

        def span_cond(st):
            return st[0] < b

        def span_body(st):
            pos, j, acc, mx = st
            send = seg_bound(j + 1)
            j_end = jnp.minimum(send, b)
            lo = pos - start_c
            hi = j_end - start_c
            n8 = lax.div(hi - lo, 8)

            def oct_body(q, c):
                acc, mx = c
                i0 = lo + 8 * q
                d = None
                for k in range(NCHUNK):
                    sl = pl.ds(k * LANES, LANES)
                    xk = [fbuf[i0 + u, sl] * wv[k] for u in range(8)]
                    if d is None:
                        d = xk
                    else:
                        d = [d[u] + xk[u] for u in range(8)]
                g = [gate_of(d[u]) for u in range(8)]
                nacc = []
                nmx = []
                for k in range(NCHUNK):
                    sl = pl.ds(k * LANES, LANES)
                    x = [fbuf[i0 + u, sl] for u in range(8)]
                    y = [x[u] * g[u] for u in range(8)]
                    s = ((y[0] + y[1]) + (y[2] + y[3])) + \
                        ((y[4] + y[5]) + (y[6] + y[7]))
                    m = jnp.maximum(
                        jnp.maximum(jnp.maximum(x[0], x[1]),
                                    jnp.maximum(x[2], x[3])),
                        jnp.maximum(jnp.maximum(x[4], x[5]),
                                    jnp.maximum(x[6], x[7])))
                    nacc.append(acc[k] + s)
                    nmx.append(jnp.maximum(mx[k], m))
                return tuple(nacc), tuple(nmx)

            acc, mx = lax.fori_loop(0, n8, oct_body, (acc, mx))
            acc, mx = lax.fori_loop(lo + 8 * n8, hi, row_at, (acc, mx))
            finished = send <= b

            @pl.when(finished)
            def _():
                row = j - seg_base
                for k in range(NCHUNK):
                    sl = pl.ds(k * LANES, LANES)
                    tsum[row, sl] = acc[k]
                    tmax[row, sl] = mx[k]

            acc = tuple(jnp.where(finished, zero, acc[k])
                        for k in range(NCHUNK))
            mx = tuple(jnp.where(finished, ninf, mx[k])
                       for k in range(NCHUNK))
            j = jnp.where(finished, j + 1, j)
            return j_end, j, acc, mx

        j, acc, mx = carry
        _, j, acc, mx = lax.while_loop(span_cond, span_body, (a, j, acc, mx))
        return j, acc, mx

    bufs = ((fbuf0, sf0), (fbuf1, sf1), (fbuf2, sf2), (fbuf3, sf3))

    for i in range(NBUF):
        @pl.when(i < ntiles)
        def _(i=i):
            feat_copy(i, *bufs[i]).start()

    def ring_body(h, carry):
        for p in range(NBUF):
            t = h * NBUF + p

            @pl.when(t < ntiles)
            def _():
                feat_copy(t, *bufs[p]).wait()

            carry = process_tile(t, bufs[p][0], carry)
            nxt = t + NBUF

            @pl.when(nxt < ntiles)
            def _():
                feat_copy(nxt, *bufs[p]).start()
        return carry

    init = (seg_base,
            tuple(zero for _ in range(NCHUNK)),
            tuple(ninf for _ in range(NCHUNK)))
    ngrps = lax.div(ntiles + NBUF - 1, NBUF)
    lax.fori_loop(0, ngrps, ring_body, init)

    pltpu.sync_copy(tsum, out_sum_hbm.at[pl.ds(seg_base, SEGS_PER_WORKER)])
    pltpu.sync_copy(tmax, out_max_hbm.at[pl.ds(seg_base, SEGS_PER_WORKER)])


@jax.jit
def _run(feats, wvec, bvec, bounds):
    mesh = plsc.VectorSubcoreMesh(
        core_axis_name="c", subcore_axis_name="s",
        num_cores=NUM_CORES, num_subcores=NUM_SUBCORES)
    fn = pl.kernel(
        _sc_body,
        out_type=[
            jax.ShapeDtypeStruct((NUM_SEGMENTS, FEATS), jnp.float32),
            jax.ShapeDtypeStruct((NUM_SEGMENTS, FEATS), jnp.float32),
        ],
        mesh=mesh,
        scratch_types=(
            [pltpu.VMEM((TILE, FEATS), jnp.float32)] * NBUF
            + [
                pltpu.VMEM((FEATS,), jnp.float32),
                pltpu.VMEM((LANES,), jnp.float32),
                pltpu.VMEM((NBOUNDS,), jnp.int32),
                pltpu.VMEM((SEGS_PER_WORKER, FEATS), jnp.float32),
                pltpu.VMEM((SEGS_PER_WORKER, FEATS), jnp.float32),
            ]
            + [pltpu.SemaphoreType.DMA] * NBUF
        ),
        compiler_params=pltpu.CompilerParams(needs_layout_passes=False),
    )
    out_sum, out_max = fn(feats, wvec, bvec, bounds)
    return jnp.concatenate([out_sum, out_max], axis=1)


def kernel(feats, segment_ids, W, b):
    ids32 = segment_ids.astype(jnp.int32)
    probes = jnp.arange(NUM_SEGMENTS + 1, dtype=jnp.int32)
    bounds = jnp.searchsorted(ids32, probes, side="left").astype(jnp.int32)
    bounds = jnp.pad(bounds, (0, NBOUNDS - bounds.shape[0]),
                     constant_values=feats.shape[0])
    wvec = W.reshape(FEATS).astype(jnp.float32)
    bvec = jnp.broadcast_to(b.reshape(()), (LANES,)).astype(jnp.float32)
    return _run(feats, wvec, bvec, bounds)


# speedup vs baseline: 1.4022x; 1.4022x over previous
import jax
import jax.numpy as jnp
from jax import lax
from jax.experimental import pallas as pl
from jax.experimental.pallas import tpu as pltpu
from jax.experimental.pallas import tpu_sc as plsc

NUM_SEGMENTS = 512
FEATS = 128
LANES = 16
NCHUNK = FEATS // LANES
NUM_CORES = 2
NUM_SUBCORES = 16
NUM_WORKERS = NUM_CORES * NUM_SUBCORES
SEGS_PER_WORKER = NUM_SEGMENTS // NUM_WORKERS
TILE = 224
NBOUNDS = 544
GATE_BLK = 8192
NBUF = 4


def _gate_body(f_ref, w_ref, b_ref, o_ref):
    z = jnp.dot(f_ref[...], w_ref[...], preferred_element_type=jnp.float32)
    g = 1.0 / (1.0 + jnp.exp(-(z + b_ref[...])))
    o_ref[...] = g.reshape(GATE_BLK // FEATS, FEATS)


def _sc_body(feats_hbm, gates_hbm, bounds_hbm, out_sum_hbm, out_max_hbm,
             fbuf0, fbuf1, fbuf2, fbuf3, gbuf0, gbuf1, gbuf2, gbuf3,
             bndbuf, tsum, tmax,
             sf0, sf1, sf2, sf3, sg0, sg1, sg2, sg3):
    n_rows = feats_hbm.shape[0]
    wid = lax.axis_index("c") * NUM_SUBCORES + lax.axis_index("s")
    seg_base = wid * SEGS_PER_WORKER

    pltpu.sync_copy(bounds_hbm, bndbuf)

    def seg_bound(j):
        return bndbuf[pl.ds(j, LANES)][0]

    zero = jnp.zeros((LANES,), jnp.float32)
    ninf = jnp.full((LANES,), -jnp.inf, jnp.float32)

    def init_row(i, _):
        for k in range(NCHUNK):
            tsum[i, pl.ds(k * LANES, LANES)] = zero
            tmax[i, pl.ds(k * LANES, LANES)] = ninf
        return 0

    lax.fori_loop(0, SEGS_PER_WORKER, init_row, 0)

    r0 = seg_bound(seg_base)
    r1 = seg_bound(seg_base + SEGS_PER_WORKER)
    r0a = (r0 // 8) * 8
    ntiles = lax.div(r1 - r0a + TILE - 1, TILE)

    def tile_start(t):
        start = r0a + t * TILE
        return start, jnp.minimum(start, n_rows - TILE)

    def copies(t, fb, gb, sf, sg):
        _, start_c = tile_start(t)
        cf = pltpu.make_async_copy(
            feats_hbm.at[pl.ds(start_c, TILE)], fb, sf)
        cg = pltpu.make_async_copy(
            gates_hbm.at[pl.ds(start_c, TILE)], gb.at[pl.ds(0, TILE)], sg)
        return cf, cg

    def issue(t, fb, gb, sf, sg):
        cf, cg = copies(t, fb, gb, sf, sg)
        cf.start()
        cg.start()

    def wait(t, fb, gb, sf, sg):
        cf, cg = copies(t, fb, gb, sf, sg)
        cf.wait()
        cg.wait()

    def process_tile(t, fbuf, gbuf, carry):
        start, start_c = tile_start(t)
        a = jnp.maximum(r0, start)
        b = jnp.minimum(r1, start + TILE)

        def row_at(i, g, acc, mx):
            x = [fbuf[i, pl.ds(k * LANES, LANES)] for k in range(NCHUNK)]
            nacc = tuple(acc[k] + x[k] * g for k in range(NCHUNK))
            nmx = tuple(jnp.maximum(mx[k], x[k]) for k in range(NCHUNK))
            return nacc, nmx

        def row_body(i, c):
            acc, mx = c
            return row_at(i, gbuf[pl.ds(i, LANES)][0], acc, mx)

        def span_cond(st):
            return st[0] < b

        def span_body(st):
            pos, j, acc, mx = st
            send = seg_bound(j + 1)
            j_end = jnp.minimum(send, b)
            lo = pos - start_c
            hi = j_end - start_c
            n8 = lax.div(hi - lo, 8)

            def oct_body(q, c):
                acc, mx = c
                i0 = lo + 8 * q
                gv = gbuf[pl.ds(i0, LANES)]
                g = [gv[u] for u in range(8)]
                nacc = []
                nmx = []
                for k in range(NCHUNK):
                    sl = pl.ds(k * LANES, LANES)
                    x = [fbuf[i0 + u, sl] for u in range(8)]
                    y = [x[u] * g[u] for u in range(8)]
                    s = ((y[0] + y[1]) + (y[2] + y[3])) + \
                        ((y[4] + y[5]) + (y[6] + y[7]))
                    m = jnp.maximum(
                        jnp.maximum(jnp.maximum(x[0], x[1]),
                                    jnp.maximum(x[2], x[3])),
                        jnp.maximum(jnp.maximum(x[4], x[5]),
                                    jnp.maximum(x[6], x[7])))
                    nacc.append(acc[k] + s)
                    nmx.append(jnp.maximum(mx[k], m))
                return tuple(nacc), tuple(nmx)

            acc, mx = lax.fori_loop(0, n8, oct_body, (acc, mx))
            acc, mx = lax.fori_loop(lo + 8 * n8, hi, row_body, (acc, mx))
            finished = send <= b

            @pl.when(finished)
            def _():
                row = j - seg_base
                for k in range(NCHUNK):
                    sl = pl.ds(k * LANES, LANES)
                    tsum[row, sl] = acc[k]
                    tmax[row, sl] = mx[k]

            acc = tuple(jnp.where(finished, zero, acc[k])
                        for k in range(NCHUNK))
            mx = tuple(jnp.where(finished, ninf, mx[k])
                       for k in range(NCHUNK))
            j = jnp.where(finished, j + 1, j)
            return j_end, j, acc, mx

        j, acc, mx = carry
        _, j, acc, mx = lax.while_loop(span_cond, span_body, (a, j, acc, mx))
        return j, acc, mx

    bufs = ((fbuf0, gbuf0, sf0, sg0), (fbuf1, gbuf1, sf1, sg1),
            (fbuf2, gbuf2, sf2, sg2), (fbuf3, gbuf3, sf3, sg3))

    for i in range(NBUF):
        @pl.when(i < ntiles)
        def _(i=i):
            issue(i, *bufs[i])

    def ring_body(h, carry):
        for p in range(NBUF):
            t = h * NBUF + p

            @pl.when(t < ntiles)
            def _():
                wait(t, *bufs[p])

            carry = process_tile(t, bufs[p][0], bufs[p][1], carry)
            nxt = t + NBUF

            @pl.when(nxt < ntiles)
            def _():
                issue(nxt, *bufs[p])
        return carry

    init = (seg_base,
            tuple(zero for _ in range(NCHUNK)),
            tuple(ninf for _ in range(NCHUNK)))
    ngrps = lax.div(ntiles + NBUF - 1, NBUF)
    lax.fori_loop(0, ngrps, ring_body, init)

    pltpu.sync_copy(tsum, out_sum_hbm.at[pl.ds(seg_base, SEGS_PER_WORKER)])
    pltpu.sync_copy(tmax, out_max_hbm.at[pl.ds(seg_base, SEGS_PER_WORKER)])


@jax.jit
def _run(feats, w2d, b2d, bounds):
    n = feats.shape[0]
    ngrid = pl.cdiv(n, GATE_BLK)
    rows_per_blk = GATE_BLK // FEATS
    gates = pl.pallas_call(
        _gate_body,
        grid=(ngrid,),
        in_specs=[
            pl.BlockSpec((GATE_BLK, FEATS), lambda i: (i, 0)),
            pl.BlockSpec((FEATS, 1), lambda i: (0, 0)),
            pl.BlockSpec((1, 1), lambda i: (0, 0)),
        ],
        out_specs=pl.BlockSpec((rows_per_blk, FEATS), lambda i: (i, 0)),
        out_shape=jax.ShapeDtypeStruct((ngrid * rows_per_blk, FEATS),
                                       jnp.float32),
    )(feats, w2d, b2d).reshape(ngrid * GATE_BLK)

    mesh = plsc.VectorSubcoreMesh(
        core_axis_name="c", subcore_axis_name="s",
        num_cores=NUM_CORES, num_subcores=NUM_SUBCORES)
    fn = pl.kernel(
        _sc_body,
        out_type=[
            jax.ShapeDtypeStruct((NUM_SEGMENTS, FEATS), jnp.float32),
            jax.ShapeDtypeStruct((NUM_SEGMENTS, FEATS), jnp.float32),
        ],
        mesh=mesh,
        scratch_types=(
            [pltpu.VMEM((TILE, FEATS), jnp.float32)] * NBUF
            + [pltpu.VMEM((TILE + LANES,), jnp.float32)] * NBUF
            + [
                pltpu.VMEM((NBOUNDS,), jnp.int32),
                pltpu.VMEM((SEGS_PER_WORKER, FEATS), jnp.float32),
                pltpu.VMEM((SEGS_PER_WORKER, FEATS), jnp.float32),
            ]
            + [pltpu.SemaphoreType.DMA] * (2 * NBUF)
        ),
        compiler_params=pltpu.CompilerParams(needs_layout_passes=False),
    )
    out_sum, out_max = fn(feats, gates, bounds)
    return jnp.concatenate([out_sum, out_max], axis=1)


def kernel(feats, segment_ids, W, b):
    ids32 = segment_ids.astype(jnp.int32)
    probes = jnp.arange(NUM_SEGMENTS + 1, dtype=jnp.int32)
    bounds = jnp.searchsorted(ids32, probes, side="left").astype(jnp.int32)
    bounds = jnp.pad(bounds, (0, NBOUNDS - bounds.shape[0]), constant_values=feats.shape[0])
    w2d = W.reshape(FEATS, 1).astype(jnp.float32)
    b2d = b.reshape(1, 1).astype(jnp.float32)
    return _run(feats, w2d, b2d, bounds)


# speedup vs baseline: 1.4137x; 1.0082x over previous
import jax
import jax.numpy as jnp
from jax import lax
from jax.experimental import pallas as pl
from jax.experimental.pallas import tpu as pltpu
from jax.experimental.pallas import tpu_sc as plsc

NUM_SEGMENTS = 512
FEATS = 128
LANES = 16
NCHUNK = FEATS // LANES
NUM_CORES = 2
NUM_SUBCORES = 16
NUM_WORKERS = NUM_CORES * NUM_SUBCORES
SEGS_PER_WORKER = NUM_SEGMENTS // NUM_WORKERS
TILE = 224
NBOUNDS = 544
GATE_BLK = 16384
NBUF = 4


def _gate_body(f_ref, w_ref, b_ref, o_ref):
    z = jnp.dot(f_ref[...], w_ref[...], preferred_element_type=jnp.float32)
    g = 1.0 / (1.0 + jnp.exp(-(z + b_ref[...])))
    o_ref[...] = g.reshape(GATE_BLK // FEATS, FEATS)


def _sc_body(feats_hbm, gates_hbm, bounds_hbm, out_sum_hbm, out_max_hbm,
             fbuf0, fbuf1, fbuf2, fbuf3, gbuf0, gbuf1, gbuf2, gbuf3,
             bndbuf, tsum, tmax,
             sf0, sf1, sf2, sf3, sg0, sg1, sg2, sg3):
    n_rows = feats_hbm.shape[0]
    wid = lax.axis_index("c") * NUM_SUBCORES + lax.axis_index("s")
    seg_base = wid * SEGS_PER_WORKER

    pltpu.sync_copy(bounds_hbm, bndbuf)

    def seg_bound(j):
        return bndbuf[pl.ds(j, LANES)][0]

    zero = jnp.zeros((LANES,), jnp.float32)
    ninf = jnp.full((LANES,), -jnp.inf, jnp.float32)

    def init_row(i, _):
        for k in range(NCHUNK):
            tsum[i, pl.ds(k * LANES, LANES)] = zero
            tmax[i, pl.ds(k * LANES, LANES)] = ninf
        return 0

    lax.fori_loop(0, SEGS_PER_WORKER, init_row, 0)

    r0 = seg_bound(seg_base)
    r1 = seg_bound(seg_base + SEGS_PER_WORKER)
    r0a = (r0 // 8) * 8
    ntiles = lax.div(r1 - r0a + TILE - 1, TILE)

    def tile_start(t):
        start = r0a + t * TILE
        return start, jnp.minimum(start, n_rows - TILE)

    def copies(t, fb, gb, sf, sg):
        _, start_c = tile_start(t)
        cf = pltpu.make_async_copy(
            feats_hbm.at[pl.ds(start_c, TILE)], fb, sf)
        cg = pltpu.make_async_copy(
            gates_hbm.at[pl.ds(start_c, TILE)], gb.at[pl.ds(0, TILE)], sg)
        return cf, cg

    def issue(t, fb, gb, sf, sg):
        cf, cg = copies(t, fb, gb, sf, sg)
        cf.start()
        cg.start()

    def wait(t, fb, gb, sf, sg):
        cf, cg = copies(t, fb, gb, sf, sg)
        cf.wait()
        cg.wait()

    def process_tile(t, fbuf, gbuf, carry):
        start, start_c = tile_start(t)
        a = jnp.maximum(r0, start)
        b = jnp.minimum(r1, start + TILE)

        def row_at(i, g, acc, mx):
            x = [fbuf[i, pl.ds(k * LANES, LANES)] for k in range(NCHUNK)]
            nacc = tuple(acc[k] + x[k] * g for k in range(NCHUNK))
            nmx = tuple(jnp.maximum(mx[k], x[k]) for k in range(NCHUNK))
            return nacc, nmx

        def row_body(i, c):
            acc, mx = c
            return row_at(i, gbuf[pl.ds(i, LANES)][0], acc, mx)

        def span_cond(st):
            return st[0] < b

        def span_body(st):
            pos, j, acc, mx = st
            send = seg_bound(j + 1)
            j_end = jnp.minimum(send, b)
            lo = pos - start_c
            hi = j_end - start_c
            n8 = lax.div(hi - lo, 8)

            def oct_body(q, c):
                acc, mx = c
                i0 = lo + 8 * q
                gv = gbuf[pl.ds(i0, LANES)]
                g = [gv[u] for u in range(8)]
                nacc = []
                nmx = []
                for k in range(NCHUNK):
                    sl = pl.ds(k * LANES, LANES)
                    x = [fbuf[i0 + u, sl] for u in range(8)]
                    y = [x[u] * g[u] for u in range(8)]
                    s = ((y[0] + y[1]) + (y[2] + y[3])) + \
                        ((y[4] + y[5]) + (y[6] + y[7]))
                    m = jnp.maximum(
                        jnp.maximum(jnp.maximum(x[0], x[1]),
                                    jnp.maximum(x[2], x[3])),
                        jnp.maximum(jnp.maximum(x[4], x[5]),
                                    jnp.maximum(x[6], x[7])))
                    nacc.append(acc[k] + s)
                    nmx.append(jnp.maximum(mx[k], m))
                return tuple(nacc), tuple(nmx)

            acc, mx = lax.fori_loop(0, n8, oct_body, (acc, mx))
            acc, mx = lax.fori_loop(lo + 8 * n8, hi, row_body, (acc, mx))
            finished = send <= b

            @pl.when(finished)
            def _():
                row = j - seg_base
                for k in range(NCHUNK):
                    sl = pl.ds(k * LANES, LANES)
                    tsum[row, sl] = acc[k]
                    tmax[row, sl] = mx[k]

            acc = tuple(jnp.where(finished, zero, acc[k])
                        for k in range(NCHUNK))
            mx = tuple(jnp.where(finished, ninf, mx[k])
                       for k in range(NCHUNK))
            j = jnp.where(finished, j + 1, j)
            return j_end, j, acc, mx

        j, acc, mx = carry
        _, j, acc, mx = lax.while_loop(span_cond, span_body, (a, j, acc, mx))
        return j, acc, mx

    bufs = ((fbuf0, gbuf0, sf0, sg0), (fbuf1, gbuf1, sf1, sg1),
            (fbuf2, gbuf2, sf2, sg2), (fbuf3, gbuf3, sf3, sg3))

    for i in range(NBUF):
        @pl.when(i < ntiles)
        def _(i=i):
            issue(i, *bufs[i])

    def ring_body(h, carry):
        for p in range(NBUF):
            t = h * NBUF + p

            @pl.when(t < ntiles)
            def _():
                wait(t, *bufs[p])

            carry = process_tile(t, bufs[p][0], bufs[p][1], carry)
            nxt = t + NBUF

            @pl.when(nxt < ntiles)
            def _():
                issue(nxt, *bufs[p])
        return carry

    init = (seg_base,
            tuple(zero for _ in range(NCHUNK)),
            tuple(ninf for _ in range(NCHUNK)))
    ngrps = lax.div(ntiles + NBUF - 1, NBUF)
    lax.fori_loop(0, ngrps, ring_body, init)

    pltpu.sync_copy(tsum, out_sum_hbm.at[pl.ds(seg_base, SEGS_PER_WORKER)])
    pltpu.sync_copy(tmax, out_max_hbm.at[pl.ds(seg_base, SEGS_PER_WORKER)])


@jax.jit
def _run(feats, w2d, b2d, bounds):
    n = feats.shape[0]
    ngrid = pl.cdiv(n, GATE_BLK)
    rows_per_blk = GATE_BLK // FEATS
    gates = pl.pallas_call(
        _gate_body,
        grid=(ngrid,),
        in_specs=[
            pl.BlockSpec((GATE_BLK, FEATS), lambda i: (i, 0)),
            pl.BlockSpec((FEATS, 1), lambda i: (0, 0)),
            pl.BlockSpec((1, 1), lambda i: (0, 0)),
        ],
        out_specs=pl.BlockSpec((rows_per_blk, FEATS), lambda i: (i, 0)),
        out_shape=jax.ShapeDtypeStruct((ngrid * rows_per_blk, FEATS),
                                       jnp.float32),
    )(feats, w2d, b2d).reshape(ngrid * GATE_BLK)

    mesh = plsc.VectorSubcoreMesh(
        core_axis_name="c", subcore_axis_name="s",
        num_cores=NUM_CORES, num_subcores=NUM_SUBCORES)
    fn = pl.kernel(
        _sc_body,
        out_type=[
            jax.ShapeDtypeStruct((NUM_SEGMENTS, FEATS), jnp.float32),
            jax.ShapeDtypeStruct((NUM_SEGMENTS, FEATS), jnp.float32),
        ],
        mesh=mesh,
        scratch_types=(
            [pltpu.VMEM((TILE, FEATS), jnp.float32)] * NBUF
            + [pltpu.VMEM((TILE + LANES,), jnp.float32)] * NBUF
            + [
                pltpu.VMEM((NBOUNDS,), jnp.int32),
                pltpu.VMEM((SEGS_PER_WORKER, FEATS), jnp.float32),
                pltpu.VMEM((SEGS_PER_WORKER, FEATS), jnp.float32),
            ]
            + [pltpu.SemaphoreType.DMA] * (2 * NBUF)
        ),
        compiler_params=pltpu.CompilerParams(needs_layout_passes=False),
    )
    out_sum, out_max = fn(feats, gates, bounds)
    return jnp.concatenate([out_sum, out_max], axis=1)


def kernel(feats, segment_ids, W, b):
    ids32 = segment_ids.astype(jnp.int32)
    probes = jnp.arange(NUM_SEGMENTS + 1, dtype=jnp.int32)
    bounds = jnp.searchsorted(ids32, probes, side="left").astype(jnp.int32)
    bounds = jnp.pad(bounds, (0, NBOUNDS - bounds.shape[0]), constant_values=feats.shape[0])
    w2d = W.reshape(FEATS, 1).astype(jnp.float32)
    b2d = b.reshape(1, 1).astype(jnp.float32)
    return _run(feats, w2d, b2d, bounds)
